# baseline (speedup 1.0000x reference)
import functools

import jax
import jax.numpy as jnp
from jax import lax
from jax.experimental import pallas as pl
from jax.experimental.pallas import tpu as pltpu
from jax.experimental.pallas import tpu_sc as plsc

K = 10
L = 20
LPAD = 24
LANES = 16
CHUNK = 128


def kernel(op1, op2, sub_table, borrow_table):
    del sub_table, borrow_table
    B = op1.shape[0]
    a_t = jnp.transpose(op1, (2, 1, 0))
    b_t = jnp.transpose(op2, (2, 1, 0))

    info = plsc.get_sparse_core_info()
    nw = info.num_cores * info.num_subcores
    cols_per_w = B // nw
    n_chunks = cols_per_w // CHUNK
    assert cols_per_w % CHUNK == 0 and B % nw == 0

    mesh = plsc.VectorSubcoreMesh(core_axis_name="c", subcore_axis_name="s")

    @functools.partial(
        pl.kernel,
        mesh=mesh,
        compiler_params=pltpu.CompilerParams(needs_layout_passes=False),
        out_type=jax.ShapeDtypeStruct((K, L, B), jnp.float32),
        scratch_types=[
            pltpu.VMEM((K, LPAD, CHUNK), jnp.float32),
            pltpu.VMEM((K, LPAD, CHUNK), jnp.float32),
            pltpu.VMEM((K, LPAD, CHUNK), jnp.float32),
            pltpu.VMEM((K, LPAD, CHUNK), jnp.float32),
            pltpu.SemaphoreType.DMA,
            pltpu.SemaphoreType.DMA,
            pltpu.SemaphoreType.DMA,
            pltpu.SemaphoreType.DMA,
            pltpu.SemaphoreType.DMA,
            pltpu.SemaphoreType.DMA,
        ],
    )
    def sc_k(a_hbm, b_hbm, out_hbm, a0, a1, b0, b1,
             sa0, sa1, sb0, sb1, so0, so1):
        wid = lax.axis_index("s") * info.num_cores + lax.axis_index("c")
        base_w = wid * cols_per_w
        av = (a0, a1)
        bv = (b0, b1)
        sa = (sa0, sa1)
        sb = (sb0, sb1)
        so = (so0, so1)

        def copy_a_in(c, bi):
            return pltpu.async_copy(
                a_hbm.at[:, :, pl.ds(base_w + c * CHUNK, CHUNK)],
                av[bi].at[:, pl.ds(0, L), :], sa[bi])

        def copy_b_in(c, bi):
            return pltpu.async_copy(
                b_hbm.at[:, :, pl.ds(base_w + c * CHUNK, CHUNK)],
                bv[bi].at[:, pl.ds(0, L), :], sb[bi])

        ins_a = [copy_a_in(0, 0), copy_a_in(1, 1)]
        ins_b = [copy_b_in(0, 0), copy_b_in(1, 1)]
        outs = [None, None]

        for c in range(n_chunks):
            cur = c % 2
            a_v = av[cur]
            b_v = bv[cur]
            ins_a[cur].wait()
            ins_b[cur].wait()

            @plsc.parallel_loop(0, CHUNK // LANES, unroll=2)
            def group_body(g):
                lane0 = g * LANES

                def step(i, carry):
                    bd0, bd1 = carry
                    ax = [a_v[x, i, pl.ds(lane0, LANES)] for x in range(K)]
                    by = [b_v[y, i, pl.ds(lane0, LANES)] for y in range(K)]
                    r = []
                    for v in range(K):
                        acc = ax[0] * by[(0 - v) % K]
                        for x in range(1, K):
                            acc = acc + ax[x] * by[(x - v) % K]
                        r.append(acc)
                    for v in range(K):
                        res = bd0 * r[v] + bd1 * r[(v + 1) % K]
                        a_v[v, i, pl.ds(lane0, LANES)] = res
                    run = ax[0]
                    p = by[1] * run
                    for y in range(2, K):
                        run = run + ax[y - 1]
                        p = p + by[y] * run
                    q = p + r[0]
                    sa = run + ax[K - 1]
                    sb = by[0]
                    for y in range(1, K):
                        sb = sb + by[y]
                    s = sa * sb * (bd0 + bd1)
                    nb1 = bd0 * p + bd1 * q
                    nb0 = s - nb1
                    return nb0, nb1

                lax.fori_loop(
                    0, L, step,
                    (jnp.ones((LANES,), jnp.float32),
                     jnp.zeros((LANES,), jnp.float32)),
                    unroll=2,
                )

            outs[cur] = pltpu.async_copy(
                a_v.at[:, pl.ds(0, L), :],
                out_hbm.at[:, :, pl.ds(base_w + c * CHUNK, CHUNK)], so[cur])
            if c + 2 < n_chunks:
                ins_b[cur] = copy_b_in(c + 2, cur)
                outs[cur].wait()
                ins_a[cur] = copy_a_in(c + 2, cur)

        for p in outs:
            if p is not None:
                p.wait()

    out_t = sc_k(a_t, b_t)
    return jnp.transpose(out_t, (2, 1, 0))


# speedup vs baseline: 1.2033x; 1.2033x over previous
import functools

import jax
import jax.numpy as jnp
from jax import lax
from jax.experimental import pallas as pl
from jax.experimental.pallas import tpu as pltpu
from jax.experimental.pallas import tpu_sc as plsc

K = 10
L = 20
LPAD = 24
LANES = 16
CHUNK = 128


def kernel(op1, op2, sub_table, borrow_table):
    del sub_table, borrow_table
    B = op1.shape[0]
    a_t = jnp.transpose(op1, (2, 1, 0))
    b_t = jnp.transpose(op2, (2, 1, 0))

    info = plsc.get_sparse_core_info()
    nw = info.num_cores * info.num_subcores
    cols_per_w = B // nw
    n_chunks = cols_per_w // CHUNK
    assert cols_per_w % CHUNK == 0 and B % nw == 0

    mesh = plsc.VectorSubcoreMesh(core_axis_name="c", subcore_axis_name="s")

    @functools.partial(
        pl.kernel,
        mesh=mesh,
        compiler_params=pltpu.CompilerParams(needs_layout_passes=False),
        out_type=jax.ShapeDtypeStruct((K, L, B), jnp.float32),
        scratch_types=[
            pltpu.VMEM((K, LPAD, CHUNK), jnp.float32),
            pltpu.VMEM((K, LPAD, CHUNK), jnp.float32),
            pltpu.VMEM((K, LPAD, CHUNK), jnp.float32),
            pltpu.VMEM((K, LPAD, CHUNK), jnp.float32),
            pltpu.SemaphoreType.DMA,
            pltpu.SemaphoreType.DMA,
            pltpu.SemaphoreType.DMA,
            pltpu.SemaphoreType.DMA,
            pltpu.SemaphoreType.DMA,
            pltpu.SemaphoreType.DMA,
        ],
    )
    def sc_k(a_hbm, b_hbm, out_hbm, a0, a1, b0, b1,
             sa0, sa1, sb0, sb1, so0, so1):
        wid = lax.axis_index("s") * info.num_cores + lax.axis_index("c")
        base_w = wid * cols_per_w
        av = (a0, a1)
        bv = (b0, b1)
        sa = (sa0, sa1)
        sb = (sb0, sb1)
        so = (so0, so1)

        def copy_a_in(c, bi):
            return pltpu.async_copy(
                a_hbm.at[:, :, pl.ds(base_w + c * CHUNK, CHUNK)],
                av[bi].at[:, pl.ds(0, L), :], sa[bi])

        def copy_b_in(c, bi):
            return pltpu.async_copy(
                b_hbm.at[:, :, pl.ds(base_w + c * CHUNK, CHUNK)],
                bv[bi].at[:, pl.ds(0, L), :], sb[bi])

        ins_a = [copy_a_in(0, 0), copy_a_in(1, 1)]
        ins_b = [copy_b_in(0, 0), copy_b_in(1, 1)]
        outs = [None, None]

        for c in range(n_chunks):
            cur = c % 2
            a_v = av[cur]
            b_v = bv[cur]
            ins_a[cur].wait()
            ins_b[cur].wait()

            def group_body(g, _):
                lane0 = g * LANES

                def step(i, carry):
                    bd0, bd1 = carry
                    ax = [a_v[x, i, pl.ds(lane0, LANES)] for x in range(K)]
                    by = [b_v[y, i, pl.ds(lane0, LANES)] for y in range(K)]
                    run = ax[0]
                    p = by[1] * run
                    for y in range(2, K):
                        run = run + ax[y - 1]
                        p = p + by[y] * run
                    h = K // 2
                    bt = [by[(-u) % K] for u in range(K)]
                    ap = [ax[j] + ax[j + h] for j in range(h)]
                    am = [ax[j] - ax[j + h] for j in range(h)]
                    bp = [bt[j] + bt[j + h] for j in range(h)]
                    bm = [bt[j] - bt[j + h] for j in range(h)]
                    cp = [None] * h
                    cm = [None] * h
                    for ii in range(h):
                        for jj in range(h):
                            m = (ii + jj) % h
                            tp = ap[ii] * bp[jj]
                            cp[m] = tp if cp[m] is None else cp[m] + tp
                            tm = am[ii] * bm[jj]
                            if cm[m] is None:
                                cm[m] = tm if ii + jj < h else -tm
                            elif ii + jj < h:
                                cm[m] = cm[m] + tm
                            else:
                                cm[m] = cm[m] - tm
                    hb0 = 0.5 * bd0
                    hb1 = 0.5 * bd1
                    r = [None] * K
                    for j in range(h):
                        r[j] = cp[j] + cm[j]
                        r[j + h] = cp[j] - cm[j]
                    for v in range(K):
                        res = hb0 * r[v] + hb1 * r[(v + 1) % K]
                        a_v[v, i, pl.ds(lane0, LANES)] = res
                    d0 = 0.5 * r[0]
                    q = p + d0
                    sa = run + ax[K - 1]
                    sb = by[0]
                    for y in range(1, K):
                        sb = sb + by[y]
                    s = sa * sb * (bd0 + bd1)
                    nb1 = bd0 * p + bd1 * q
                    nb0 = s - nb1
                    return nb0, nb1

                lax.fori_loop(
                    0, L, step,
                    (jnp.ones((LANES,), jnp.float32),
                     jnp.zeros((LANES,), jnp.float32)),
                    unroll=2,
                )
                return 0

            lax.fori_loop(0, CHUNK // LANES, group_body, 0)
            outs[cur] = pltpu.async_copy(
                a_v.at[:, pl.ds(0, L), :],
                out_hbm.at[:, :, pl.ds(base_w + c * CHUNK, CHUNK)], so[cur])
            if c + 2 < n_chunks:
                ins_b[cur] = copy_b_in(c + 2, cur)
                outs[cur].wait()
                ins_a[cur] = copy_a_in(c + 2, cur)

        for p in outs:
            if p is not None:
                p.wait()

    out_t = sc_k(a_t, b_t)
    return jnp.transpose(out_t, (2, 1, 0))


# speedup vs baseline: 1.2647x; 1.0511x over previous
import functools

import jax
import jax.numpy as jnp
from jax import lax
from jax.experimental import pallas as pl
from jax.experimental.pallas import tpu as pltpu
from jax.experimental.pallas import tpu_sc as plsc

K = 10
L = 20
LPAD = 24
LANES = 16
CHUNK = 128


def kernel(op1, op2, sub_table, borrow_table):
    del sub_table, borrow_table
    B = op1.shape[0]
    a_t = jnp.transpose(op1, (2, 1, 0))
    b_t = jnp.transpose(op2, (2, 1, 0))

    info = plsc.get_sparse_core_info()
    nw = info.num_cores * info.num_subcores
    cols_per_w = B // nw
    n_chunks = cols_per_w // CHUNK
    assert cols_per_w % CHUNK == 0 and B % nw == 0

    mesh = plsc.VectorSubcoreMesh(core_axis_name="c", subcore_axis_name="s")

    @functools.partial(
        pl.kernel,
        mesh=mesh,
        compiler_params=pltpu.CompilerParams(needs_layout_passes=False),
        out_type=jax.ShapeDtypeStruct((K, L, B), jnp.float32),
        scratch_types=[
            pltpu.VMEM((K, LPAD, CHUNK), jnp.float32),
            pltpu.VMEM((K, LPAD, CHUNK), jnp.float32),
            pltpu.VMEM((K, LPAD, CHUNK), jnp.float32),
            pltpu.VMEM((K, LPAD, CHUNK), jnp.float32),
            pltpu.SemaphoreType.DMA,
            pltpu.SemaphoreType.DMA,
            pltpu.SemaphoreType.DMA,
            pltpu.SemaphoreType.DMA,
            pltpu.SemaphoreType.DMA,
            pltpu.SemaphoreType.DMA,
        ],
    )
    def sc_k(a_hbm, b_hbm, out_hbm, a0, a1, b0, b1,
             sa0, sa1, sb0, sb1, so0, so1):
        wid = lax.axis_index("s") * info.num_cores + lax.axis_index("c")
        base_w = wid * cols_per_w
        av = (a0, a1)
        bv = (b0, b1)
        sa = (sa0, sa1)
        sb = (sb0, sb1)
        so = (so0, so1)

        def copy_a_in(c, bi):
            return pltpu.async_copy(
                a_hbm.at[:, :, pl.ds(base_w + c * CHUNK, CHUNK)],
                av[bi].at[:, pl.ds(0, L), :], sa[bi])

        def copy_b_in(c, bi):
            return pltpu.async_copy(
                b_hbm.at[:, :, pl.ds(base_w + c * CHUNK, CHUNK)],
                bv[bi].at[:, pl.ds(0, L), :], sb[bi])

        ins_a = [copy_a_in(0, 0), None]
        ins_b = [copy_b_in(0, 0), None]
        outs = [None, None]

        for c in range(n_chunks):
            cur = c % 2
            nxt = 1 - cur
            a_v = av[cur]
            b_v = bv[cur]
            ins_a[cur].wait()
            ins_b[cur].wait()

            def group_body(g, _):
                lane0 = g * LANES

                def step(i, carry):
                    bd0, bd1 = carry
                    ax = [a_v[x, i, pl.ds(lane0, LANES)] for x in range(K)]
                    by = [b_v[y, i, pl.ds(lane0, LANES)] for y in range(K)]
                    run = ax[0]
                    p = by[1] * run
                    for y in range(2, K):
                        run = run + ax[y - 1]
                        p = p + by[y] * run
                    h = K // 2
                    bt = [by[(-u) % K] for u in range(K)]
                    ap = [ax[j] + ax[j + h] for j in range(h)]
                    am = [ax[j] - ax[j + h] for j in range(h)]
                    bp = [bt[j] + bt[j + h] for j in range(h)]
                    bm = [bt[j] - bt[j + h] for j in range(h)]
                    cp = [None] * h
                    cm = [None] * h
                    for ii in range(h):
                        for jj in range(h):
                            m = (ii + jj) % h
                            tp = ap[ii] * bp[jj]
                            cp[m] = tp if cp[m] is None else cp[m] + tp
                            tm = am[ii] * bm[jj]
                            if cm[m] is None:
                                cm[m] = tm if ii + jj < h else -tm
                            elif ii + jj < h:
                                cm[m] = cm[m] + tm
                            else:
                                cm[m] = cm[m] - tm
                    hb0 = 0.5 * bd0
                    hb1 = 0.5 * bd1
                    r = [None] * K
                    for j in range(h):
                        r[j] = cp[j] + cm[j]
                        r[j + h] = cp[j] - cm[j]
                    for v in range(K):
                        res = hb0 * r[v] + hb1 * r[(v + 1) % K]
                        a_v[v, i, pl.ds(lane0, LANES)] = res
                    d0 = 0.5 * r[0]
                    q = p + d0
                    sa = run + ax[K - 1]
                    sb = by[0]
                    for y in range(1, K):
                        sb = sb + by[y]
                    s = sa * sb * (bd0 + bd1)
                    nb1 = bd0 * p + bd1 * q
                    nb0 = s - nb1
                    return nb0, nb1

                lax.fori_loop(
                    0, L, step,
                    (jnp.ones((LANES,), jnp.float32),
                     jnp.zeros((LANES,), jnp.float32)),
                    unroll=2,
                )
                return 0

            half = CHUNK // LANES // 2
            lax.fori_loop(0, half, group_body, 0)
            if c + 1 < n_chunks:
                ins_b[nxt] = copy_b_in(c + 1, nxt)
                if outs[nxt] is not None:
                    outs[nxt].wait()
                ins_a[nxt] = copy_a_in(c + 1, nxt)
            lax.fori_loop(half, CHUNK // LANES, group_body, 0)
            outs[cur] = pltpu.async_copy(
                a_v.at[:, pl.ds(0, L), :],
                out_hbm.at[:, :, pl.ds(base_w + c * CHUNK, CHUNK)], so[cur])

        for p in outs:
            if p is not None:
                p.wait()

    out_t = sc_k(a_t, b_t)
    return jnp.transpose(out_t, (2, 1, 0))
